# Initial kernel scaffold; baseline (speedup 1.0000x reference)
import jax
import jax.numpy as jnp
from jax.experimental import pallas as pl


def kernel(gt_region, pred_region, gt_affinity, pred_affinity):
    raise NotImplementedError("write your pallas kernel here")


# speedup vs baseline: 19.9149x; 19.9149x over previous
import functools

import jax
import jax.numpy as jnp
from jax import lax
from jax.experimental import pallas as pl
from jax.experimental.pallas import tpu as pltpu
from jax.experimental.pallas import tpu_sc as plsc

B, H, W = 16, 512, 512
N = B * H * W
VEC = 16
NTILES = 16
PER_TILE = N // NTILES
CHUNK = 8192
NCHUNKS = PER_TILE // CHUNK
HB = 2048
NCH = HB // VEC
STATS = 128
ROW = 2 * HB + STATS


def _sc_score_map_loss(gtr, prr, gta, pra):
    mesh = plsc.VectorSubcoreMesh(core_axis_name="c", subcore_axis_name="s")

    @functools.partial(
        pl.kernel,
        mesh=mesh,
        compiler_params=pltpu.CompilerParams(needs_layout_passes=False),
        out_type=jax.ShapeDtypeStruct((2, 128), jnp.float32),
        scratch_types=[
            pltpu.VMEM((CHUNK,), jnp.float32),
            pltpu.VMEM((CHUNK,), jnp.float32),
            pltpu.VMEM((HB,), jnp.float32),
            pltpu.VMEM((HB,), jnp.float32),
            pltpu.VMEM((ROW,), jnp.float32),
            pltpu.VMEM((ROW,), jnp.float32),
            pltpu.VMEM_SHARED((NTILES, ROW), jnp.float32),
        ],
    )
    def launch(gtr_h, prr_h, gta_h, pra_h, out_h,
               gt_v, pr_v, hc_v, hs_v, row_v, acc_v, board):
        cid = lax.axis_index("c")
        sid = lax.axis_index("s")
        zeros = jnp.zeros((VEC,), jnp.float32)
        ones = jnp.full((VEC,), 1.0, jnp.float32)

        def zero_hists(i, carry):
            hc_v[pl.ds(i * VEC, VEC)] = zeros
            hs_v[pl.ds(i * VEC, VEC)] = zeros
            return carry

        lax.fori_loop(0, NCH, zero_hists, 0)

        def zero_stats(i, carry):
            row_v[pl.ds(i * VEC, VEC)] = zeros
            return carry

        lax.fori_loop(0, STATS // VEC, zero_stats, 0)

        def run(gt_h, pr_h):
            base = sid * PER_TILE

            def chunk_body(it, carry):
                off = base + it * CHUNK
                pltpu.sync_copy(gt_h.at[pl.ds(off, CHUNK)], gt_v)
                pltpu.sync_copy(pr_h.at[pl.ds(off, CHUNK)], pr_v)

                def vec_body(j, c2):
                    pacc, nacc = c2
                    g = gt_v[pl.ds(j * VEC, VEC)]
                    p = pr_v[pl.ds(j * VEC, VEC)]
                    d = g - p
                    v = d * d
                    pv = v * g
                    nv = v - pv
                    bi = jnp.minimum((nv * float(HB)).astype(jnp.int32), HB - 1)
                    plsc.addupdate_scatter(hc_v, [bi], ones)
                    plsc.addupdate_scatter(hs_v, [bi], nv)
                    return (pacc + pv, nacc + g)

                return lax.fori_loop(0, CHUNK // VEC, vec_body, carry)

            pacc, nacc = lax.fori_loop(0, NCHUNKS, chunk_body, (zeros, zeros))
            row_v[pl.ds(0, VEC)] = pacc
            row_v[pl.ds(VEC, VEC)] = nacc

        @pl.when(cid == 0)
        def _():
            run(gtr_h, prr_h)

        @pl.when(cid == 1)
        def _():
            run(gta_h, pra_h)

        pltpu.sync_copy(hc_v, board.at[sid, pl.ds(0, HB)])
        pltpu.sync_copy(hs_v, board.at[sid, pl.ds(HB, HB)])
        pltpu.sync_copy(row_v.at[pl.ds(0, STATS)],
                        board.at[sid, pl.ds(2 * HB, STATS)])
        plsc.subcore_barrier()

        @pl.when(sid == 0)
        def _():
            def zero_acc(i, carry):
                acc_v[pl.ds(i * VEC, VEC)] = zeros
                return carry

            lax.fori_loop(0, ROW // VEC, zero_acc, 0)

            def add_tile(t, carry):
                pltpu.sync_copy(board.at[t], row_v)

                def add_vec(j, c2):
                    sl = pl.ds(j * VEC, VEC)
                    acc_v[sl] = acc_v[sl] + row_v[sl]
                    return c2

                return lax.fori_loop(0, ROW // VEC, add_vec, carry)

            lax.fori_loop(0, NTILES, add_tile, 0)

            psum = jnp.sum(acc_v[pl.ds(2 * HB, VEC)])
            npos = jnp.sum(acc_v[pl.ds(2 * HB + VEC, VEC)])
            k = 3.0 * npos

            def scan_body(j, carry):
                cab, sfull, spart, stot = carry
                i = NCH - 1 - j
                rc = lax.rev(acc_v[pl.ds(i * VEC, VEC)], (0,))
                rs = lax.rev(acc_v[pl.ds(HB + i * VEC, VEC)], (0,))
                ic = plsc.cumsum(rc) + cab
                ec = ic - rc
                sfull = sfull + jnp.sum(jnp.where(ic <= k, rs, 0.0))
                part = (k - ec) * rs / jnp.maximum(rc, 1.0)
                on_boundary = (ic > k) & (ec < k)
                spart = spart + jnp.sum(jnp.where(on_boundary, part, 0.0))
                return (cab + jnp.sum(rc), sfull, spart, stot + jnp.sum(rs))

            ctot, sfull, spart, stot = lax.fori_loop(
                0, NCH, scan_body, (0.0, 0.0, 0.0, 0.0))
            nneg = ctot - npos
            tot_neg = jnp.where(nneg >= k, sfull + spart, stot)
            res_vec = (jnp.full((VEC,), psum, jnp.float32)
                       / jnp.full((VEC,), npos, jnp.float32)
                       + jnp.full((VEC,), tot_neg, jnp.float32))

            def fill_out(i, carry):
                row_v[pl.ds(i * VEC, VEC)] = res_vec
                return carry

            lax.fori_loop(0, 128 // VEC, fill_out, 0)
            pltpu.sync_copy(row_v.at[pl.ds(0, 128)], out_h.at[cid])

    return launch(gtr, prr, gta, pra)


def kernel(gt_region, pred_region, gt_affinity, pred_affinity):
    out = _sc_score_map_loss(
        gt_region.reshape(-1), pred_region.reshape(-1),
        gt_affinity.reshape(-1), pred_affinity.reshape(-1))
    return out[0, 0] + out[1, 0]


# speedup vs baseline: 57.4380x; 2.8842x over previous
import functools

import jax
import jax.numpy as jnp
from jax import lax
from jax.experimental import pallas as pl
from jax.experimental.pallas import tpu as pltpu
from jax.experimental.pallas import tpu_sc as plsc

B, H, W = 16, 512, 512
N = B * H * W
VEC = 16
NTILES = 16
PER_TILE = N // NTILES
CHUNK = 16384
NBUF = 2
NCHUNKS = PER_TILE // CHUNK
UNROLL = 8
HB = 2048
NCH = HB // VEC
HISTP = 2 * HB + 128
POSCH = (HISTP - HB) // VEC
ROWP = 10240
SLICE = ROWP // NTILES
RVEC = SLICE // VEC


def _sc_score_map_loss(gtr, prr, gta, pra):
    mesh = plsc.VectorSubcoreMesh(core_axis_name="c", subcore_axis_name="s")

    @functools.partial(
        pl.kernel,
        mesh=mesh,
        compiler_params=pltpu.CompilerParams(needs_layout_passes=False),
        out_type=jax.ShapeDtypeStruct((2, 128), jnp.float32),
        scratch_types=[
            pltpu.VMEM((NBUF * CHUNK,), jnp.float32),
            pltpu.VMEM((NBUF * CHUNK,), jnp.float32),
            pltpu.VMEM((HISTP,), jnp.float32),
            pltpu.VMEM((HISTP,), jnp.float32),
            pltpu.VMEM((SLICE,), jnp.float32),
            pltpu.VMEM((SLICE,), jnp.float32),
            pltpu.VMEM((128,), jnp.float32),
            pltpu.SemaphoreType.DMA,
            pltpu.SemaphoreType.DMA,
            pltpu.SemaphoreType.DMA,
            pltpu.SemaphoreType.DMA,
            pltpu.VMEM_SHARED(((NTILES + 1) * ROWP,), jnp.float32),
        ],
    )
    def launch(gtr_h, prr_h, gta_h, pra_h, out_h,
               gt_v, pr_v, hc_v, hs_v, red_v, acc_v, outb_v,
               sg0, sg1, sp0, sp1, board):
        cid = lax.axis_index("c")
        sid = lax.axis_index("s")
        zeros = jnp.zeros((VEC,), jnp.float32)
        ones = jnp.full((VEC,), 1.0, jnp.float32)
        hbf = jnp.full((VEC,), float(HB), jnp.float32)
        sg = (sg0, sg1)
        sp = (sp0, sp1)

        def zero_hists(i, carry):
            hc_v[pl.ds(i * VEC, VEC)] = zeros
            hs_v[pl.ds(i * VEC, VEC)] = zeros
            return carry

        lax.fori_loop(0, HISTP // VEC, zero_hists, 0)

        def run(gt_h, pr_h):
            base = sid * PER_TILE
            for b in range(NBUF):
                off = base + b * CHUNK
                dst = pl.ds(b * CHUNK, CHUNK)
                pltpu.async_copy(gt_h.at[pl.ds(off, CHUNK)], gt_v.at[dst], sg[b])
                pltpu.async_copy(pr_h.at[pl.ds(off, CHUNK)], pr_v.at[dst], sp[b])

            def outer(i, carry):
                for b in range(NBUF):
                    c = i * NBUF + b
                    dst = pl.ds(b * CHUNK, CHUNK)
                    pltpu.make_async_copy(
                        gt_h.at[pl.ds(0, CHUNK)], gt_v.at[dst], sg[b]).wait()
                    pltpu.make_async_copy(
                        pr_h.at[pl.ds(0, CHUNK)], pr_v.at[dst], sp[b]).wait()

                    @plsc.parallel_loop(0, CHUNK // VEC, unroll=UNROLL)
                    def _(j):
                        o = b * CHUNK + j * VEC
                        g = gt_v[pl.ds(o, VEC)]
                        p = pr_v[pl.ds(o, VEC)]
                        d = g - p
                        v = d * d
                        key = (g + p) * hbf
                        bi = key.astype(jnp.int32)
                        plsc.addupdate_scatter(hc_v, [bi], ones)
                        plsc.addupdate_scatter(hs_v, [bi], v)

                    @pl.when(c + NBUF < NCHUNKS)
                    def _():
                        off = base + (c + NBUF) * CHUNK
                        pltpu.async_copy(
                            gt_h.at[pl.ds(off, CHUNK)], gt_v.at[dst], sg[b])
                        pltpu.async_copy(
                            pr_h.at[pl.ds(off, CHUNK)], pr_v.at[dst], sp[b])
                return carry

            lax.fori_loop(0, NCHUNKS // NBUF, outer, 0)

        @pl.when(cid == 0)
        def _():
            run(gtr_h, prr_h)

        @pl.when(cid == 1)
        def _():
            run(gta_h, pra_h)

        row0 = sid * ROWP
        pltpu.sync_copy(hc_v, board.at[pl.ds(row0, HISTP)])
        pltpu.sync_copy(hs_v, board.at[pl.ds(row0 + HISTP, HISTP)])
        plsc.subcore_barrier()

        def zero_acc(i, carry):
            acc_v[pl.ds(i * VEC, VEC)] = zeros
            return carry

        lax.fori_loop(0, RVEC, zero_acc, 0)

        def add_tile(t, carry):
            pltpu.sync_copy(board.at[pl.ds(t * ROWP + sid * SLICE, SLICE)], red_v)

            def add_vec(j, c2):
                sl = pl.ds(j * VEC, VEC)
                acc_v[sl] = acc_v[sl] + red_v[sl]
                return c2

            return lax.fori_loop(0, RVEC, add_vec, carry)

        lax.fori_loop(0, NTILES, add_tile, 0)
        pltpu.sync_copy(acc_v,
                        board.at[pl.ds(NTILES * ROWP + sid * SLICE, SLICE)])
        plsc.subcore_barrier()

        @pl.when(sid == 0)
        def _():
            pltpu.sync_copy(board.at[pl.ds(NTILES * ROWP, HISTP)], hc_v)
            pltpu.sync_copy(board.at[pl.ds(NTILES * ROWP + HISTP, HISTP)], hs_v)

            def pos_body(j, c2):
                pc, ps = c2
                sl = pl.ds(HB + j * VEC, VEC)
                return (pc + hc_v[sl], ps + hs_v[sl])

            pcnt, psum_v = lax.fori_loop(0, POSCH, pos_body, (zeros, zeros))
            npos = jnp.sum(pcnt)
            psum = jnp.sum(psum_v)
            k = 3.0 * npos

            def scan_body(j, carry):
                cab, sfull, spart, stot = carry
                i = NCH - 1 - j
                rc = lax.rev(hc_v[pl.ds(i * VEC, VEC)], (0,))
                rs = lax.rev(hs_v[pl.ds(i * VEC, VEC)], (0,))
                ic = plsc.cumsum(rc) + cab
                ec = ic - rc
                sfull = sfull + jnp.sum(jnp.where(ic <= k, rs, 0.0))
                part = (k - ec) * rs / jnp.maximum(rc, 1.0)
                on_boundary = (ic > k) & (ec < k)
                spart = spart + jnp.sum(jnp.where(on_boundary, part, 0.0))
                return (cab + jnp.sum(rc), sfull, spart, stot + jnp.sum(rs))

            nneg, sfull, spart, stot = lax.fori_loop(
                0, NCH, scan_body, (0.0, 0.0, 0.0, 0.0))
            tot_neg = jnp.where(nneg >= k, sfull + spart, stot)
            res_vec = (jnp.full((VEC,), psum, jnp.float32)
                       / jnp.full((VEC,), npos, jnp.float32)
                       + jnp.full((VEC,), tot_neg, jnp.float32))

            def fill_out(i, carry):
                outb_v[pl.ds(i * VEC, VEC)] = res_vec
                return carry

            lax.fori_loop(0, 128 // VEC, fill_out, 0)
            pltpu.sync_copy(outb_v, out_h.at[cid])

    return launch(gtr, prr, gta, pra)


def kernel(gt_region, pred_region, gt_affinity, pred_affinity):
    out = _sc_score_map_loss(
        gt_region.reshape(-1), pred_region.reshape(-1),
        gt_affinity.reshape(-1), pred_affinity.reshape(-1))
    return out[0, 0] + out[1, 0]
